# baseline (speedup 1.0000x reference)
import functools

import jax
import jax.numpy as jnp
from jax import lax
from jax.experimental import pallas as pl
from jax.experimental.pallas import tpu as pltpu
from jax.experimental.pallas import tpu_sc as plsc

N = 10000
E = 320000
D = 128
NG = 128
DO = 16

NC, NS = 2, 16
NW = NC * NS
NP_ = 10240
TPR = NP_ // NW
CAP = 11008
SCH = 2000
PCH = 256

BLK = 1000
NB = N // BLK
HALF = 5
TPB = BLK // 8
CBLK = 640
NCB = NP_ // CBLK

def _mesh():
    return plsc.VectorSubcoreMesh(core_axis_name="c", subcore_axis_name="s")


PB = 256
NPB = E // PB

def _pos_core(dst_ref, pos_ref, cnt_ref, base_ref):
    i = pl.program_id(0)

    @pl.when(i == 0)
    def _():
        base_ref[...] = jnp.zeros((1, D), jnp.float32)

    tid = jnp.floor(dst_ref[...].astype(jnp.float32) / float(TPR)).astype(jnp.int32)
    gids = lax.broadcasted_iota(jnp.int32, (PB, D), 1)
    onehot = jnp.where(tid == gids, 1.0, 0.0)
    tri = jnp.where(
        lax.broadcasted_iota(jnp.int32, (PB, PB), 0)
        >= lax.broadcasted_iota(jnp.int32, (PB, PB), 1), 1.0, 0.0)
    ranks = lax.dot_general(tri, onehot, (((1,), (0,)), ((), ())),
                            preferred_element_type=jnp.float32,
                            precision=lax.Precision.HIGHEST)
    slot0 = (lax.broadcasted_iota(jnp.int32, (1, D), 1).astype(jnp.float32) * float(CAP))
    sel = onehot * (ranks + base_ref[...] + slot0)
    pos = jnp.sum(sel, axis=1, keepdims=True) - 1.0
    pos_ref[...] = pos.astype(jnp.int32)
    base_ref[...] += jnp.sum(onehot, axis=0, keepdims=True)

    @pl.when(i == NPB - 1)
    def _():
        cnt_ref[...] = base_ref[...].astype(jnp.int32)


def _pos_body(dst_ref, pos_ref, cnt_ref, base_ref):
    _pos_core(dst_ref, pos_ref, cnt_ref, base_ref)


def _positions(dstp):
    return pl.pallas_call(
        _pos_body,
        grid=(NPB,),
        in_specs=[pl.BlockSpec((PB, 1), lambda i: (i, 0))],
        out_specs=[
            pl.BlockSpec((PB, 1), lambda i: (i, 0)),
            pl.BlockSpec((1, D), lambda i: (0, 0)),
        ],
        out_shape=[
            jax.ShapeDtypeStruct((E, 1), jnp.int32),
            jax.ShapeDtypeStruct((1, D), jnp.int32),
        ],
        scratch_shapes=[pltpu.VMEM((1, D), jnp.float32)],
    )(dstp[:, None])


ECH = 128
NECH = E // ECH

def _scatter_lists(srcp, dstp, pos):
    @functools.partial(
        pl.kernel,
        out_type=[
            jax.ShapeDtypeStruct((NW * CAP,), jnp.int32),
            jax.ShapeDtypeStruct((NW * CAP,), jnp.int32),
        ],
        mesh=_mesh(),
        scratch_types=[
            pltpu.VMEM((ECH,), jnp.int32),
            pltpu.VMEM((ECH,), jnp.int32),
            pltpu.VMEM((ECH,), jnp.int32),
        ],
    )
    def scat(src_hbm, dst_hbm, pos_hbm, osrc, odst, pb, sb, db):
        cid = lax.axis_index("c")
        sid = lax.axis_index("s")
        wid = cid * NS + sid
        trips = (NECH // NW) + jnp.where(wid < (NECH % NW), 1, 0)

        def chunk(ci, carry):
            base = (wid + ci * NW) * ECH
            pltpu.sync_copy(pos_hbm.at[pl.ds(base, ECH)], pb)
            pltpu.sync_copy(src_hbm.at[pl.ds(base, ECH)], sb)
            pltpu.sync_copy(dst_hbm.at[pl.ds(base, ECH)], db)
            pltpu.sync_copy(sb, osrc.at[pb])
            pltpu.sync_copy(db, odst.at[pb])
            return carry

        lax.fori_loop(0, trips, chunk, 0)

    return scat(srcp, dstp, pos)


RPS = NP_ // NC

def _agg_call(h, csrc, cdst, cnts, zrs):
    @functools.partial(
        pl.kernel,
        out_type=jax.ShapeDtypeStruct((NP_, D), jnp.float32),
        mesh=_mesh(),
        scratch_types=[
            pltpu.VMEM((ECH,), jnp.int32),
            pltpu.VMEM((ECH,), jnp.int32),
            pltpu.VMEM((D + 16,), jnp.int32),
            pltpu.VMEM((ECH, D), jnp.float32),
            pltpu.VMEM_SHARED((RPS, D), jnp.float32),
            pltpu.SemaphoreType.DMA,
        ],
    )
    def agg(h_hbm, cs_hbm, cd_hbm, cnt_hbm, z_hbm, out_hbm,
            srcb, dstb, cntb, rows, acc, sem):
        cid = lax.axis_index("c")
        sid = lax.axis_index("s")
        wid = cid * NS + sid
        lo = wid * TPR
        scb = cid * RPS
        pltpu.sync_copy(cnt_hbm, cntb.at[pl.ds(0, D)])
        cnt = cntb[pl.ds(wid, 16)][0]
        pltpu.sync_copy(z_hbm, acc.at[pl.ds(sid * TPR, TPR)])
        plsc.subcore_barrier()
        iot = lax.iota(jnp.int32, 16)

        def chunk(ci, carry):
            base = wid * CAP + ci * ECH
            pltpu.sync_copy(cs_hbm.at[pl.ds(base, ECH)], srcb)
            pltpu.sync_copy(cd_hbm.at[pl.ds(base, ECH)], dstb)
            for g in range(ECH // 16):
                sl = pl.ds(g * 16, 16)
                sv = srcb[sl]
                dv = dstb[sl]
                ok = ((ci * ECH + g * 16 + iot) < cnt)
                ok = ok & (dv >= lo) & (dv < lo + TPR) & (sv >= 0) & (sv <= N)
                srcb[sl] = jnp.where(ok, sv, N)
                dstb[sl] = jnp.where(ok, dv, lo) - scb
            pltpu.async_copy(h_hbm.at[srcb], rows, sem).wait()
            pltpu.sync_copy(rows, acc.at[dstb], add=True)
            return carry

        lax.fori_loop(0, CAP // ECH, chunk, 0)
        plsc.subcore_barrier()
        pltpu.sync_copy(acc.at[pl.ds(sid * TPR, TPR)], out_hbm.at[pl.ds(lo, TPR)])

    return agg(h, csrc, cdst, cnts, zrs)


def _fold(acc):
    a4 = acc[0:4] + acc[4:8]
    a2 = a4[0:2] + a4[2:4]
    return a2[0:1] + a2[1:2]


def _acc_halves(y_ref, accA_ref, accB_ref, i):
    @pl.when(i == 0)
    def _():
        accA_ref[...] = jnp.zeros((8, D), jnp.float32)
        accB_ref[...] = jnp.zeros((8, D), jnp.float32)

    def body(t, a):
        return a + y_ref[pl.ds(t * 8, 8), :]

    @pl.when(i < HALF)
    def _():
        accA_ref[...] = lax.fori_loop(0, TPB, body, accA_ref[...])

    @pl.when(i >= HALF)
    def _():
        accB_ref[...] = lax.fori_loop(0, TPB, body, accB_ref[...])


def _emit_sum(accA_ref, accB_ref, s_ref, i):
    @pl.when(i == NB - 1)
    def _():
        s_ref[...] = _fold(accA_ref[...]) + _fold(accB_ref[...])


_RN = float(jnp.float32(1.0) / jnp.float32(N))


def _a_body(h_ref, agg_ref, w_ref, b_ref, y_ref, s_ref, accA, accB):
    i = pl.program_id(0)
    m = h_ref[...] + agg_ref[...]
    y = jnp.dot(m, w_ref[...], preferred_element_type=jnp.float32) + b_ref[...]
    y_ref[...] = y
    _acc_halves(y_ref, accA, accB, i)
    _emit_sum(accA, accB, s_ref, i)


def _stage_a(h, agg, w, b):
    return pl.pallas_call(
        _a_body,
        grid=(NB,),
        in_specs=[
            pl.BlockSpec((BLK, D), lambda i: (i, 0)),
            pl.BlockSpec((BLK, D), lambda i: (i, 0)),
            pl.BlockSpec((D, D), lambda i: (0, 0)),
            pl.BlockSpec((1, D), lambda i: (0, 0)),
        ],
        out_specs=[
            pl.BlockSpec((BLK, D), lambda i: (i, 0)),
            pl.BlockSpec((1, D), lambda i: (0, 0)),
        ],
        out_shape=[
            jax.ShapeDtypeStruct((NP_, D), jnp.float32),
            jax.ShapeDtypeStruct((1, D), jnp.float32),
        ],
        scratch_shapes=[
            pltpu.VMEM((8, D), jnp.float32),
            pltpu.VMEM((8, D), jnp.float32),
        ],
    )(h, agg, w, b)


def _v_body(y_ref, s_ref, v_ref, sq_ref, accA, accB):
    i = pl.program_id(0)
    mean = s_ref[...] * _RN
    d = y_ref[...] - mean
    sq_ref[...] = d * d
    _acc_halves(sq_ref, accA, accB, i)

    @pl.when(i == NB - 1)
    def _():
        v_ref[...] = (_fold(accA[...]) + _fold(accB[...])) * _RN


def _stage_v(y, s):
    return pl.pallas_call(
        _v_body,
        grid=(NB,),
        in_specs=[
            pl.BlockSpec((BLK, D), lambda i: (i, 0)),
            pl.BlockSpec((1, D), lambda i: (0, 0)),
        ],
        out_specs=pl.BlockSpec((1, D), lambda i: (0, 0)),
        out_shape=jax.ShapeDtypeStruct((1, D), jnp.float32),
        scratch_shapes=[
            pltpu.VMEM((BLK, D), jnp.float32),
            pltpu.VMEM((8, D), jnp.float32),
            pltpu.VMEM((8, D), jnp.float32),
        ],
    )(y, s)


def _b_body(y1_ref, s_ref, v_ref, g_ref, bb_ref, w_ref, b_ref,
            y2_ref, s2_ref, accA, accB):
    i = pl.program_id(0)
    mean = s_ref[...] * _RN
    sd = jnp.sqrt(v_ref[...] + 1e-5)
    xn = (y1_ref[...] - mean) / sd * g_ref[...] + bb_ref[...]
    xr = jnp.maximum(xn, 0.0)
    y2 = jnp.dot(xr, w_ref[...], preferred_element_type=jnp.float32) + b_ref[...]
    y2_ref[...] = y2
    _acc_halves(y2_ref, accA, accB, i)
    _emit_sum(accA, accB, s2_ref, i)


def _stage_b(y1, s1, v1, g, bb, w2, b2):
    return pl.pallas_call(
        _b_body,
        grid=(NB,),
        in_specs=[
            pl.BlockSpec((BLK, D), lambda i: (i, 0)),
            pl.BlockSpec((1, D), lambda i: (0, 0)),
            pl.BlockSpec((1, D), lambda i: (0, 0)),
            pl.BlockSpec((1, D), lambda i: (0, 0)),
            pl.BlockSpec((1, D), lambda i: (0, 0)),
            pl.BlockSpec((D, D), lambda i: (0, 0)),
            pl.BlockSpec((1, D), lambda i: (0, 0)),
        ],
        out_specs=[
            pl.BlockSpec((BLK, D), lambda i: (i, 0)),
            pl.BlockSpec((1, D), lambda i: (0, 0)),
        ],
        out_shape=[
            jax.ShapeDtypeStruct((NP_, D), jnp.float32),
            jax.ShapeDtypeStruct((1, D), jnp.float32),
        ],
        scratch_shapes=[
            pltpu.VMEM((8, D), jnp.float32),
            pltpu.VMEM((8, D), jnp.float32),
        ],
    )(y1, s1, v1, g, bb, w2, b2)


def _c_body(y2_ref, s_ref, v_ref, g_ref, bb_ref, h_ref):
    i = pl.program_id(0)
    mean = s_ref[...] * _RN
    sd = jnp.sqrt(v_ref[...] + 1e-5)
    xn = (y2_ref[...] - mean) / sd * g_ref[...] + bb_ref[...]
    hv = jnp.maximum(xn, 0.0)
    rows = i * CBLK + lax.broadcasted_iota(jnp.int32, (CBLK, 1), 0)
    h_ref[...] = jnp.where(rows < N, hv, 0.0)


def _stage_c(y2, s2, v2, g, bb):
    return pl.pallas_call(
        _c_body,
        grid=(NCB,),
        in_specs=[
            pl.BlockSpec((CBLK, D), lambda i: (i, 0)),
            pl.BlockSpec((1, D), lambda i: (0, 0)),
            pl.BlockSpec((1, D), lambda i: (0, 0)),
            pl.BlockSpec((1, D), lambda i: (0, 0)),
            pl.BlockSpec((1, D), lambda i: (0, 0)),
        ],
        out_specs=pl.BlockSpec((CBLK, D), lambda i: (i, 0)),
        out_shape=jax.ShapeDtypeStruct((NP_, D), jnp.float32),
    )(y2, s2, v2, g, bb)


def _pool_body(h_ref, b_ref, ps_ref, cnt_ref):
    i = pl.program_id(0)
    gids = lax.broadcasted_iota(jnp.int32, (BLK, NG), 1)
    onehot = jnp.where(b_ref[...] == gids, 1.0, 0.0)
    ps = lax.dot_general(onehot, h_ref[...], (((0,), (0,)), ((), ())),
                         preferred_element_type=jnp.float32,
                         precision=lax.Precision.HIGHEST)
    c = jnp.sum(onehot, axis=0)[:, None]

    @pl.when(i == 0)
    def _():
        ps_ref[...] = ps
        cnt_ref[...] = c

    @pl.when(i > 0)
    def _():
        ps_ref[...] += ps
        cnt_ref[...] += c


def _pool(h, bp):
    return pl.pallas_call(
        _pool_body,
        grid=(NB,),
        in_specs=[
            pl.BlockSpec((BLK, D), lambda i: (i, 0)),
            pl.BlockSpec((BLK, 1), lambda i: (i, 0)),
        ],
        out_specs=[
            pl.BlockSpec((NG, D), lambda i: (0, 0)),
            pl.BlockSpec((NG, 1), lambda i: (0, 0)),
        ],
        out_shape=[
            jax.ShapeDtypeStruct((NG, D), jnp.float32),
            jax.ShapeDtypeStruct((NG, 1), jnp.float32),
        ],
    )(h, bp)


_RG = float(jnp.float32(1.0) / jnp.float32(NG))


def _final_body(ps_ref, cnt_ref, w1_ref, b1_ref, g_ref, bb_ref, w2_ref, b2_ref,
                o_ref, scr):
    pooled = ps_ref[...] / jnp.maximum(cnt_ref[...], 1.0)
    o = jnp.dot(pooled, w1_ref[...], preferred_element_type=jnp.float32) + b1_ref[...]
    scr[...] = o

    def body(t, a):
        return a + scr[pl.ds(t * 8, 8), :]

    s = _fold(lax.fori_loop(0, NG // 8, body, jnp.zeros((8, D), jnp.float32)))
    mean = s * _RG
    d = o - mean
    scr[...] = d * d
    v = _fold(lax.fori_loop(0, NG // 8, body, jnp.zeros((8, D), jnp.float32))) * _RG
    xn = d / jnp.sqrt(v + 1e-5) * g_ref[...] + bb_ref[...]
    xr = jnp.maximum(xn, 0.0)
    o_ref[...] = jnp.dot(xr, w2_ref[...], preferred_element_type=jnp.float32) + b2_ref[...]


def _final(ps, cnt, w1, b1, g, bb, w2, b2):
    full = lambda s: pl.BlockSpec(s, lambda: (0,) * len(s))
    return pl.pallas_call(
        _final_body,
        in_specs=[
            full((NG, D)), full((NG, 1)), full((D, D)), full((1, D)),
            full((1, D)), full((1, D)), full((D, DO)), full((1, DO)),
        ],
        out_specs=full((NG, DO)),
        out_shape=jax.ShapeDtypeStruct((NG, DO), jnp.float32),
        scratch_shapes=[pltpu.VMEM((NG, D), jnp.float32)],
    )(ps, cnt, w1, b1, g, bb, w2, b2)


def kernel(x, edge_index, batch, params):
    srcp = edge_index[0].astype(jnp.int32)
    dstp = edge_index[1].astype(jnp.int32)
    xp = jnp.zeros((NP_, D), jnp.float32).at[:N, :].set(x)
    bp = batch.astype(jnp.int32)[:, None]
    zrs = jnp.zeros((TPR, D), jnp.float32)

    pos, cnts = _positions(dstp)
    csrc, cdst = _scatter_lists(srcp, dstp, pos[:, 0])
    cnts = cnts[0].astype(jnp.int32)

    h = xp
    for p in params['layers']:
        agg = _agg_call(h, csrc, cdst, cnts, zrs)
        y1, s1 = _stage_a(h, agg, p['w1'], p['b1'][None])
        v1 = _stage_v(y1, s1)
        y2, s2 = _stage_b(y1, s1, v1, p['bn1_g'][None], p['bn1_b'][None],
                          p['w2'], p['b2'][None])
        v2 = _stage_v(y2, s2)
        h = _stage_c(y2, s2, v2, p['ng'][None], p['nb'][None])

    ps, cnt = _pool(h, bp)
    c = params['cls']
    return _final(ps, cnt, c['w1'], c['b1'][None], c['bn_g'][None],
                  c['bn_b'][None], c['w2'], c['b2'][None])
